# baseline (speedup 1.0000x reference)
import functools

import numpy as np
import jax
import jax.numpy as jnp
from jax import lax
from jax.experimental import pallas as pl
from jax.experimental.pallas import tpu as pltpu
from jax.experimental.pallas import tpu_sc as plsc

N_POINTS = 524288
N_LEVELS = 16
F_DIM = 2
T_SIZE = 2 ** 19
T_MASK = T_SIZE - 1
P1 = int(np.uint32(2654435761).view(np.int32))
P2 = int(np.uint32(805459861).view(np.int32))
BASE_RES = 16
_PLS = float(np.exp2(np.log2(2048.0 / 16.0) / (N_LEVELS - 1)))


def _levels():
    out = []
    for l in range(N_LEVELS):
        scale = np.exp2(l * np.log2(_PLS)) * BASE_RES - 1.0
        res = int(np.ceil(scale)) + 1
        out.append((float(scale), res, res ** 3 <= T_SIZE))
    return out


LEVELS = _levels()

NC, NS, LANES = 2, 16, 16
NW = NC * NS
PW = N_POINTS // NW
CHUNK = 512
NCHUNK = PW // CHUNK
GROUPS = CHUNK // LANES
NIDX = 8 * CHUNK


def _encode_body(xT, tbl, out, xyz, frA, frB,
                 i0A, i0B, i1A, i1B, f0A, f0B, f1A, f1B, acc, semA, semB):
    wid = lax.axis_index("c") * NS + lax.axis_index("s")
    base = wid * PW
    frs = (frA, frB)
    idx0s = (i0A, i0B)
    idx1s = (i1A, i1B)
    f0s = (f0A, f0B)
    f1s = (f1A, f1B)
    sems = (semA, semB)

    def compute_idx(l, pb):
        scale, res, dense = LEVELS[l]
        fr = frs[pb]
        idx0 = idx0s[pb]
        idx1 = idx1s[pb]

        def g_body(g, _):
            s = pl.ds(g * LANES, LANES)
            px = xyz[0, s] * scale + 0.5
            py = xyz[1, s] * scale + 0.5
            pz = xyz[2, s] * scale + 0.5
            ix = px.astype(jnp.int32)
            iy = py.astype(jnp.int32)
            iz = pz.astype(jnp.int32)
            fr[0, s] = px - ix.astype(jnp.float32)
            fr[1, s] = py - iy.astype(jnp.float32)
            fr[2, s] = pz - iz.astype(jnp.float32)
            if dense:
                hx = (ix, ix + 1)
                hy = (iy * res, iy * res + res)
                hz = (iz * (res * res), iz * (res * res) + res * res)
                comb = lambda a, b: a + b
            else:
                hx = (ix, ix + 1)
                hy = (iy * P1, iy * P1 + P1)
                hz = (iz * P2, iz * P2 + P2)
                comb = lambda a, b: a ^ b
            for c in range(8):
                v = (comb(comb(hx[c & 1], hy[(c >> 1) & 1]), hz[(c >> 2) & 1])
                     & T_MASK) + l * T_SIZE
                v2 = v + v
                idx0[pl.ds(c * CHUNK + g * LANES, LANES)] = v2
                idx1[pl.ds(c * CHUNK + g * LANES, LANES)] = v2 + 1
            return 0

        lax.fori_loop(0, GROUPS, g_body, 0)

    def accum(l, pb):
        fr = frs[pb]
        f0b = f0s[pb]
        f1b = f1s[pb]

        def g_body(g, _):
            s = pl.ds(g * LANES, LANES)
            fx = fr[0, s]
            fy = fr[1, s]
            fz = fr[2, s]
            gx = 1.0 - fx
            gy = 1.0 - fy
            gz = 1.0 - fz
            a0 = jnp.zeros((LANES,), jnp.float32)
            a1 = jnp.zeros((LANES,), jnp.float32)
            for c in range(8):
                w = ((fx if c & 1 else gx) * (fy if c & 2 else gy)
                     * (fz if c & 4 else gz))
                sc = pl.ds(c * CHUNK + g * LANES, LANES)
                a0 = a0 + w * f0b[sc]
                a1 = a1 + w * f1b[sc]
            acc[2 * l, s] = a0
            acc[2 * l + 1, s] = a1
            return 0

        lax.fori_loop(0, GROUPS, g_body, 0)

    def chunk_body(ci, _):
        cbase = base + ci * CHUNK
        pltpu.sync_copy(xT.at[:, pl.ds(cbase, CHUNK)], xyz)
        for l in range(N_LEVELS):
            pb = l % 2
            compute_idx(l, pb)
            d0 = pltpu.async_copy(tbl.at[idx0s[pb]], f0s[pb], sems[pb])
            d1 = pltpu.async_copy(tbl.at[idx1s[pb]], f1s[pb], sems[pb])
            d0.wait()
            d1.wait()
            accum(l, pb)
        pltpu.sync_copy(acc, out.at[:, pl.ds(cbase, CHUNK)])
        return 0

    lax.fori_loop(0, NCHUNK, chunk_body, 0)


@functools.partial(jax.jit, static_argnames=())
def _encode(xT, tbl):
    kern = pl.kernel(
        _encode_body,
        out_type=jax.ShapeDtypeStruct((2 * N_LEVELS, N_POINTS), jnp.float32),
        mesh=plsc.VectorSubcoreMesh(core_axis_name="c", subcore_axis_name="s"),
        scratch_types=[
            pltpu.VMEM((3, CHUNK), jnp.float32),
            pltpu.VMEM((3, CHUNK), jnp.float32),
            pltpu.VMEM((3, CHUNK), jnp.float32),
            pltpu.VMEM((NIDX,), jnp.int32),
            pltpu.VMEM((NIDX,), jnp.int32),
            pltpu.VMEM((NIDX,), jnp.int32),
            pltpu.VMEM((NIDX,), jnp.int32),
            pltpu.VMEM((NIDX,), jnp.float32),
            pltpu.VMEM((NIDX,), jnp.float32),
            pltpu.VMEM((NIDX,), jnp.float32),
            pltpu.VMEM((NIDX,), jnp.float32),
            pltpu.VMEM((2 * N_LEVELS, CHUNK), jnp.float32),
            pltpu.SemaphoreType.DMA,
            pltpu.SemaphoreType.DMA,
        ],
        compiler_params=pltpu.CompilerParams(
            needs_layout_passes=False, use_tc_tiling_on_sc=False),
    )
    return kern(xT, tbl)


def _softplus10(v):
    t = 10.0 * v
    return (jnp.maximum(t, 0.0) + jnp.log1p(jnp.exp(-jnp.abs(t)))) * 0.1


BN = 4096


def _mlp_body(e_ref, w0_ref, w1_ref, w2_ref, w3_ref, o_ref):
    blk = e_ref[...]
    h = _softplus10(jnp.dot(w0_ref[...], blk, preferred_element_type=jnp.float32))
    h = _softplus10(jnp.dot(w1_ref[...], h, preferred_element_type=jnp.float32))
    h = _softplus10(jnp.dot(w2_ref[...], h, preferred_element_type=jnp.float32))
    o_ref[...] = jnp.dot(w3_ref[...], h, preferred_element_type=jnp.float32)


def _mlp(enc, W0, W1, W2, W3):
    grid = (N_POINTS // BN,)
    return pl.pallas_call(
        _mlp_body,
        grid=grid,
        in_specs=[
            pl.BlockSpec((2 * N_LEVELS, BN), lambda i: (0, i)),
            pl.BlockSpec(W0.shape, lambda i: (0, 0)),
            pl.BlockSpec(W1.shape, lambda i: (0, 0)),
            pl.BlockSpec(W2.shape, lambda i: (0, 0)),
            pl.BlockSpec(W3.shape, lambda i: (0, 0)),
        ],
        out_specs=pl.BlockSpec((1, BN), lambda i: (0, i)),
        out_shape=jax.ShapeDtypeStruct((1, N_POINTS), jnp.float32),
    )(enc, W0, W1, W2, W3)


def kernel(x, table, W0, W1, W2, W3):
    xT = x.T
    tbl = table.reshape(-1)
    enc = _encode(xT, tbl)
    out = _mlp(enc, W0, W1, W2, W3)
    return out.reshape(N_POINTS, 1)


# speedup vs baseline: 4.9014x; 4.9014x over previous
import functools

import numpy as np
import jax
import jax.numpy as jnp
from jax import lax
from jax.experimental import pallas as pl
from jax.experimental.pallas import tpu as pltpu
from jax.experimental.pallas import tpu_sc as plsc

N_POINTS = 524288
N_LEVELS = 16
F_DIM = 2
T_SIZE = 2 ** 19
T_MASK = T_SIZE - 1
P1 = int(np.uint32(2654435761).view(np.int32))
P2 = int(np.uint32(805459861).view(np.int32))
BASE_RES = 16
_PLS = float(np.exp2(np.log2(2048.0 / 16.0) / (N_LEVELS - 1)))


def _levels():
    out = []
    for l in range(N_LEVELS):
        scale = np.exp2(l * np.log2(_PLS)) * BASE_RES - 1.0
        res = int(np.ceil(scale)) + 1
        out.append((float(scale), res, res ** 3 <= T_SIZE))
    return out


LEVELS = _levels()

NC, NS, LANES = 2, 16, 16
NW = NC * NS
PW = N_POINTS // NW
CHUNK = 512
NCHUNK = PW // CHUNK
GROUPS = CHUNK // LANES
NIDX = 8 * CHUNK


def _encode_body(xT, tbl, out, xyz, frA, frB,
                 idxA, idxB, fpA, fpB, acc, semA, semB):
    wid = lax.axis_index("c") * NS + lax.axis_index("s")
    base = wid * PW
    frs = (frA, frB)
    idxs = (idxA, idxB)
    fps = (fpA, fpB)
    sems = (semA, semB)

    def compute_idx(l, pb):
        scale, res, dense = LEVELS[l]
        fr = frs[pb]
        idx = idxs[pb]

        def g_body(g, _):
            s = pl.ds(g * LANES, LANES)
            px = xyz[0, s] * scale + 0.5
            py = xyz[1, s] * scale + 0.5
            pz = xyz[2, s] * scale + 0.5
            ix = px.astype(jnp.int32)
            iy = py.astype(jnp.int32)
            iz = pz.astype(jnp.int32)
            fr[0, s] = px - ix.astype(jnp.float32)
            fr[1, s] = py - iy.astype(jnp.float32)
            fr[2, s] = pz - iz.astype(jnp.float32)
            if dense:
                hx = (ix, ix + 1)
                hy = (iy * res, iy * res + res)
                hz = (iz * (res * res), iz * (res * res) + res * res)
                comb = lambda a, b: a + b
            else:
                hx = (ix, ix + 1)
                hy = (iy * P1, iy * P1 + P1)
                hz = (iz * P2, iz * P2 + P2)
                comb = lambda a, b: a ^ b
            for c in range(8):
                v = (comb(comb(hx[c & 1], hy[(c >> 1) & 1]), hz[(c >> 2) & 1])
                     & T_MASK) + l * T_SIZE
                idx[pl.ds(c * CHUNK + g * LANES, LANES)] = v
            return 0

        lax.fori_loop(0, GROUPS, g_body, 0)

    def accum(l, pb):
        fr = frs[pb]
        fpb = fps[pb]

        def g_body(g, _):
            s = pl.ds(g * LANES, LANES)
            fx = fr[0, s]
            fy = fr[1, s]
            fz = fr[2, s]
            gx = 1.0 - fx
            gy = 1.0 - fy
            gz = 1.0 - fz
            a0 = jnp.zeros((LANES,), jnp.float32)
            a1 = jnp.zeros((LANES,), jnp.float32)
            for c in range(8):
                w = ((fx if c & 1 else gx) * (fy if c & 2 else gy)
                     * (fz if c & 4 else gz))
                pk = fpb[pl.ds(c * CHUNK + g * LANES, LANES)]
                f0v, f1v = plsc.unpack(plsc.bitcast(pk, jnp.bfloat16),
                                       format=plsc.PackFormat.INTERLEAVED)
                a0 = a0 + w * f0v
                a1 = a1 + w * f1v
            acc[2 * l, s] = a0
            acc[2 * l + 1, s] = a1
            return 0

        lax.fori_loop(0, GROUPS, g_body, 0)

    def chunk_body(ci, _):
        cbase = base + ci * CHUNK
        pltpu.sync_copy(xT.at[:, pl.ds(cbase, CHUNK)], xyz)
        compute_idx(0, 0)
        desc = [None, None]
        desc[0] = pltpu.async_copy(tbl.at[idxA], fpA, semA)
        for l in range(1, N_LEVELS):
            pb = l % 2
            compute_idx(l, pb)
            desc[pb] = pltpu.async_copy(tbl.at[idxs[pb]], fps[pb], sems[pb])
            desc[1 - pb].wait()
            accum(l - 1, 1 - pb)
        desc[1].wait()
        accum(N_LEVELS - 1, 1)
        pltpu.sync_copy(acc, out.at[:, pl.ds(cbase, CHUNK)])
        return 0

    lax.fori_loop(0, NCHUNK, chunk_body, 0)


@functools.partial(jax.jit, static_argnames=())
def _encode(xT, tbl):
    kern = pl.kernel(
        _encode_body,
        out_type=jax.ShapeDtypeStruct((2 * N_LEVELS, N_POINTS), jnp.float32),
        mesh=plsc.VectorSubcoreMesh(core_axis_name="c", subcore_axis_name="s"),
        scratch_types=[
            pltpu.VMEM((3, CHUNK), jnp.float32),
            pltpu.VMEM((3, CHUNK), jnp.float32),
            pltpu.VMEM((3, CHUNK), jnp.float32),
            pltpu.VMEM((NIDX,), jnp.int32),
            pltpu.VMEM((NIDX,), jnp.int32),
            pltpu.VMEM((NIDX,), jnp.float32),
            pltpu.VMEM((NIDX,), jnp.float32),
            pltpu.VMEM((2 * N_LEVELS, CHUNK), jnp.float32),
            pltpu.SemaphoreType.DMA,
            pltpu.SemaphoreType.DMA,
        ],
        compiler_params=pltpu.CompilerParams(
            needs_layout_passes=False, use_tc_tiling_on_sc=False),
    )
    return kern(xT, tbl)


def _softplus10(v):
    t = 10.0 * v
    return (jnp.maximum(t, 0.0) + jnp.log1p(jnp.exp(-jnp.abs(t)))) * 0.1


BN = 4096


def _mlp_body(e_ref, w0_ref, w1_ref, w2_ref, w3_ref, o_ref):
    blk = e_ref[...]
    h = _softplus10(jnp.dot(w0_ref[...], blk, preferred_element_type=jnp.float32))
    h = _softplus10(jnp.dot(w1_ref[...], h, preferred_element_type=jnp.float32))
    h = _softplus10(jnp.dot(w2_ref[...], h, preferred_element_type=jnp.float32))
    o_ref[...] = jnp.dot(w3_ref[...], h, preferred_element_type=jnp.float32)


def _mlp(enc, W0, W1, W2, W3):
    grid = (N_POINTS // BN,)
    return pl.pallas_call(
        _mlp_body,
        grid=grid,
        in_specs=[
            pl.BlockSpec((2 * N_LEVELS, BN), lambda i: (0, i)),
            pl.BlockSpec(W0.shape, lambda i: (0, 0)),
            pl.BlockSpec(W1.shape, lambda i: (0, 0)),
            pl.BlockSpec(W2.shape, lambda i: (0, 0)),
            pl.BlockSpec(W3.shape, lambda i: (0, 0)),
        ],
        out_specs=pl.BlockSpec((1, BN), lambda i: (0, i)),
        out_shape=jax.ShapeDtypeStruct((1, N_POINTS), jnp.float32),
    )(enc, W0, W1, W2, W3)


def kernel(x, table, W0, W1, W2, W3):
    xT = x.T
    tbl = jax.lax.bitcast_convert_type(
        table.astype(jnp.bfloat16).reshape(N_LEVELS * T_SIZE, F_DIM),
        jnp.float32)
    enc = _encode(xT, tbl)
    out = _mlp(enc, W0, W1, W2, W3)
    return out.reshape(N_POINTS, 1)


# speedup vs baseline: 6.6028x; 1.3471x over previous
import functools

import numpy as np
import jax
import jax.numpy as jnp
from jax import lax
from jax.experimental import pallas as pl
from jax.experimental.pallas import tpu as pltpu
from jax.experimental.pallas import tpu_sc as plsc

N_POINTS = 524288
N_LEVELS = 16
F_DIM = 2
T_SIZE = 2 ** 19
T_MASK = T_SIZE - 1
P1 = int(np.uint32(2654435761).view(np.int32))
P2 = int(np.uint32(805459861).view(np.int32))
BASE_RES = 16
_PLS = float(np.exp2(np.log2(2048.0 / 16.0) / (N_LEVELS - 1)))


def _levels():
    out = []
    for l in range(N_LEVELS):
        scale = np.exp2(l * np.log2(_PLS)) * BASE_RES - 1.0
        res = int(np.ceil(scale)) + 1
        out.append((float(scale), res, res ** 3 <= T_SIZE))
    return out


LEVELS = _levels()

N_LOCAL = 3
_DN = [LEVELS[l][1] ** 3 + LEVELS[l][1] ** 2 + LEVELS[l][1] + 1
       for l in range(N_LOCAL)]
DOFF = [sum(_DN[:l]) for l in range(N_LOCAL)]
DENSE_TOTAL = -(-sum(_DN) // 8) * 8
DMA_LEVELS = list(range(N_LOCAL, N_LEVELS))

NC, NS, LANES = 2, 16, 16
NW = NC * NS
PW = N_POINTS // NW
CHUNK = 512
NCHUNK = PW // CHUNK
GROUPS = CHUNK // LANES
NIDX = 8 * CHUNK


def _encode_body(xT, tbl, dense_hbm, out, xyz, dloc, frA, frB,
                 idxA, idxB, fpA, fpB, acc, semA, semB):
    wid = lax.axis_index("c") * NS + lax.axis_index("s")
    base = wid * PW
    frs = (frA, frB)
    idxs = (idxA, idxB)
    fps = (fpA, fpB)
    sems = (semA, semB)
    pltpu.sync_copy(dense_hbm, dloc)

    def compute_idx(l, pb):
        scale, res, dense = LEVELS[l]
        fr = frs[pb]
        idx = idxs[pb]

        def g_body(g, _):
            s = pl.ds(g * LANES, LANES)
            px = xyz[0, s] * scale + 0.5
            py = xyz[1, s] * scale + 0.5
            pz = xyz[2, s] * scale + 0.5
            ix = px.astype(jnp.int32)
            iy = py.astype(jnp.int32)
            iz = pz.astype(jnp.int32)
            fr[0, s] = px - ix.astype(jnp.float32)
            fr[1, s] = py - iy.astype(jnp.float32)
            fr[2, s] = pz - iz.astype(jnp.float32)
            if dense:
                hx = (ix, ix + 1)
                hy = (iy * res, iy * res + res)
                hz = (iz * (res * res), iz * (res * res) + res * res)
                comb = lambda a, b: a + b
            else:
                hx = (ix, ix + 1)
                hy = (iy * P1, iy * P1 + P1)
                hz = (iz * P2, iz * P2 + P2)
                comb = lambda a, b: a ^ b
            for c in range(8):
                v = (comb(comb(hx[c & 1], hy[(c >> 1) & 1]), hz[(c >> 2) & 1])
                     & T_MASK) + l * T_SIZE
                idx[pl.ds(c * CHUNK + g * LANES, LANES)] = v
            return 0

        lax.fori_loop(0, GROUPS, g_body, 0)

    def accum(l, pb):
        fr = frs[pb]
        fpb = fps[pb]

        def g_body(g, _):
            s = pl.ds(g * LANES, LANES)
            fx = fr[0, s]
            fy = fr[1, s]
            fz = fr[2, s]
            gx = 1.0 - fx
            gy = 1.0 - fy
            gz = 1.0 - fz
            a0 = jnp.zeros((LANES,), jnp.float32)
            a1 = jnp.zeros((LANES,), jnp.float32)
            for c in range(8):
                w = ((fx if c & 1 else gx) * (fy if c & 2 else gy)
                     * (fz if c & 4 else gz))
                pk = fpb[pl.ds(c * CHUNK + g * LANES, LANES)]
                f0v, f1v = plsc.unpack(plsc.bitcast(pk, jnp.bfloat16),
                                       format=plsc.PackFormat.INTERLEAVED)
                a0 = a0 + w * f0v
                a1 = a1 + w * f1v
            acc[2 * l, s] = a0
            acc[2 * l + 1, s] = a1
            return 0

        lax.fori_loop(0, GROUPS, g_body, 0)

    def dense_all():
        def g_body(g, _):
            s = pl.ds(g * LANES, LANES)
            x0 = xyz[0, s]
            y0 = xyz[1, s]
            z0 = xyz[2, s]
            for l in range(N_LOCAL):
                scale, res, _ = LEVELS[l]
                px = x0 * scale + 0.5
                py = y0 * scale + 0.5
                pz = z0 * scale + 0.5
                ix = px.astype(jnp.int32)
                iy = py.astype(jnp.int32)
                iz = pz.astype(jnp.int32)
                fx = px - ix.astype(jnp.float32)
                fy = py - iy.astype(jnp.float32)
                fz = pz - iz.astype(jnp.float32)
                gx = 1.0 - fx
                gy = 1.0 - fy
                gz = 1.0 - fz
                hx = (ix + DOFF[l], ix + (DOFF[l] + 1))
                hy = (iy * res, iy * res + res)
                hz = (iz * (res * res), iz * (res * res) + res * res)
                a0 = jnp.zeros((LANES,), jnp.float32)
                a1 = jnp.zeros((LANES,), jnp.float32)
                for c in range(8):
                    w = ((fx if c & 1 else gx) * (fy if c & 2 else gy)
                         * (fz if c & 4 else gz))
                    iv = hx[c & 1] + hy[(c >> 1) & 1] + hz[(c >> 2) & 1]
                    pk = plsc.load_gather(dloc, [iv])
                    f0v, f1v = plsc.unpack(plsc.bitcast(pk, jnp.bfloat16),
                                           format=plsc.PackFormat.INTERLEAVED)
                    a0 = a0 + w * f0v
                    a1 = a1 + w * f1v
                acc[2 * l, s] = a0
                acc[2 * l + 1, s] = a1
            return 0

        lax.fori_loop(0, GROUPS, g_body, 0)

    def chunk_body(ci, _):
        cbase = base + ci * CHUNK
        pltpu.sync_copy(xT.at[:, pl.ds(cbase, CHUNK)], xyz)
        desc = [None, None]
        compute_idx(DMA_LEVELS[0], 0)
        desc[0] = pltpu.async_copy(tbl.at[idxA], fpA, semA)
        for i in range(1, len(DMA_LEVELS)):
            pb = i % 2
            compute_idx(DMA_LEVELS[i], pb)
            desc[pb] = pltpu.async_copy(tbl.at[idxs[pb]], fps[pb], sems[pb])
            if i == 1:
                dense_all()
            desc[1 - pb].wait()
            accum(DMA_LEVELS[i - 1], 1 - pb)
        last = (len(DMA_LEVELS) - 1) % 2
        desc[last].wait()
        accum(DMA_LEVELS[-1], last)
        pltpu.sync_copy(acc, out.at[:, pl.ds(cbase, CHUNK)])
        return 0

    lax.fori_loop(0, NCHUNK, chunk_body, 0)


@functools.partial(jax.jit, static_argnames=())
def _encode(xT, tbl, dense):
    kern = pl.kernel(
        _encode_body,
        out_type=jax.ShapeDtypeStruct((2 * N_LEVELS, N_POINTS), jnp.float32),
        mesh=plsc.VectorSubcoreMesh(core_axis_name="c", subcore_axis_name="s"),
        scratch_types=[
            pltpu.VMEM((3, CHUNK), jnp.float32),
            pltpu.VMEM((DENSE_TOTAL,), jnp.float32),
            pltpu.VMEM((3, CHUNK), jnp.float32),
            pltpu.VMEM((3, CHUNK), jnp.float32),
            pltpu.VMEM((NIDX,), jnp.int32),
            pltpu.VMEM((NIDX,), jnp.int32),
            pltpu.VMEM((NIDX,), jnp.float32),
            pltpu.VMEM((NIDX,), jnp.float32),
            pltpu.VMEM((2 * N_LEVELS, CHUNK), jnp.float32),
            pltpu.SemaphoreType.DMA,
            pltpu.SemaphoreType.DMA,
        ],
        compiler_params=pltpu.CompilerParams(
            needs_layout_passes=False, use_tc_tiling_on_sc=False),
    )
    return kern(xT, tbl, dense)


def _softplus10(v):
    t = 10.0 * v
    return (jnp.maximum(t, 0.0) + jnp.log1p(jnp.exp(-jnp.abs(t)))) * 0.1


BN = 4096


def _mlp_body(e_ref, w0_ref, w1_ref, w2_ref, w3_ref, o_ref):
    blk = e_ref[...]
    h = _softplus10(jnp.dot(w0_ref[...], blk, preferred_element_type=jnp.float32))
    h = _softplus10(jnp.dot(w1_ref[...], h, preferred_element_type=jnp.float32))
    h = _softplus10(jnp.dot(w2_ref[...], h, preferred_element_type=jnp.float32))
    o_ref[...] = jnp.dot(w3_ref[...], h, preferred_element_type=jnp.float32)


def _mlp(enc, W0, W1, W2, W3):
    grid = (N_POINTS // BN,)
    return pl.pallas_call(
        _mlp_body,
        grid=grid,
        in_specs=[
            pl.BlockSpec((2 * N_LEVELS, BN), lambda i: (0, i)),
            pl.BlockSpec(W0.shape, lambda i: (0, 0)),
            pl.BlockSpec(W1.shape, lambda i: (0, 0)),
            pl.BlockSpec(W2.shape, lambda i: (0, 0)),
            pl.BlockSpec(W3.shape, lambda i: (0, 0)),
        ],
        out_specs=pl.BlockSpec((1, BN), lambda i: (0, i)),
        out_shape=jax.ShapeDtypeStruct((1, N_POINTS), jnp.float32),
    )(enc, W0, W1, W2, W3)


def kernel(x, table, W0, W1, W2, W3):
    xT = x.T
    tbl = jax.lax.bitcast_convert_type(
        table.astype(jnp.bfloat16).reshape(N_LEVELS * T_SIZE, F_DIM),
        jnp.float32)
    dense = jnp.concatenate(
        [tbl[l * T_SIZE:l * T_SIZE + _DN[l]] for l in range(N_LOCAL)]
        + [jnp.zeros((DENSE_TOTAL - sum(_DN),), jnp.float32)])
    enc = _encode(xT, tbl, dense)
    out = _mlp(enc, W0, W1, W2, W3)
    return out.reshape(N_POINTS, 1)


# speedup vs baseline: 7.5323x; 1.1408x over previous
import functools

import numpy as np
import jax
import jax.numpy as jnp
from jax import lax
from jax.experimental import pallas as pl
from jax.experimental.pallas import tpu as pltpu
from jax.experimental.pallas import tpu_sc as plsc

N_POINTS = 524288
N_LEVELS = 16
F_DIM = 2
T_SIZE = 2 ** 19
T_MASK = T_SIZE - 1
P1 = int(np.uint32(2654435761).view(np.int32))
P2 = int(np.uint32(805459861).view(np.int32))
BASE_RES = 16
_PLS = float(np.exp2(np.log2(2048.0 / 16.0) / (N_LEVELS - 1)))


def _levels():
    out = []
    for l in range(N_LEVELS):
        scale = np.exp2(l * np.log2(_PLS)) * BASE_RES - 1.0
        res = int(np.ceil(scale)) + 1
        out.append((float(scale), res, res ** 3 <= T_SIZE))
    return out


LEVELS = _levels()

N_LOCAL = 3
_DN = [LEVELS[l][1] ** 3 + LEVELS[l][1] ** 2 + LEVELS[l][1] + 1
       for l in range(N_LOCAL)]
DOFF = [sum(_DN[:l]) for l in range(N_LOCAL)]
DENSE_TOTAL = -(-sum(_DN) // 8) * 8
DMA_LEVELS = list(range(N_LOCAL, N_LEVELS))

SH_LEVELS = [3, 5]


def _sh_size(l):
    scale, res, dense = LEVELS[l]
    return (res ** 3 + res ** 2 + res + 1) if dense else T_SIZE


SH_OFF = {}
_acc = 0
for _l in SH_LEVELS:
    SH_OFF[_l] = _acc
    _acc += _sh_size(_l)
SH_TOTAL = -(-_acc // 8) * 8

NC, NS, LANES = 2, 16, 16
NW = NC * NS
PW = N_POINTS // NW
CHUNK = 512
NCHUNK = PW // CHUNK
GROUPS = CHUNK // LANES
NIDX = 8 * CHUNK


def _encode_body(xT, tbl, dense_hbm, sh_hbm, out, xyz, dloc, shtab, frA, frB,
                 idxA, idxB, fpA, fpB, acc, semA, semB):
    wid = lax.axis_index("c") * NS + lax.axis_index("s")
    base = wid * PW
    frs = (frA, frB)
    idxs = (idxA, idxB)
    fps = (fpA, fpB)
    sems = (semA, semB)
    pltpu.sync_copy(dense_hbm, dloc)

    @pl.when(lax.axis_index("s") == 0)
    def _fill_shared():
        pltpu.sync_copy(sh_hbm, shtab)

    plsc.subcore_barrier()

    def compute_idx(l, pb):
        scale, res, dense = LEVELS[l]
        fr = frs[pb]
        idx = idxs[pb]

        def g_body(g, _):
            s = pl.ds(g * LANES, LANES)
            px = xyz[0, s] * scale + 0.5
            py = xyz[1, s] * scale + 0.5
            pz = xyz[2, s] * scale + 0.5
            ix = px.astype(jnp.int32)
            iy = py.astype(jnp.int32)
            iz = pz.astype(jnp.int32)
            fr[0, s] = px - ix.astype(jnp.float32)
            fr[1, s] = py - iy.astype(jnp.float32)
            fr[2, s] = pz - iz.astype(jnp.float32)
            if dense:
                hx = (ix, ix + 1)
                hy = (iy * res, iy * res + res)
                hz = (iz * (res * res), iz * (res * res) + res * res)
                comb = lambda a, b: a + b
            else:
                hx = (ix, ix + 1)
                hy = (iy * P1, iy * P1 + P1)
                hz = (iz * P2, iz * P2 + P2)
                comb = lambda a, b: a ^ b
            off = SH_OFF[l] if l in SH_OFF else l * T_SIZE
            for c in range(8):
                v = (comb(comb(hx[c & 1], hy[(c >> 1) & 1]), hz[(c >> 2) & 1])
                     & T_MASK) + off
                idx[pl.ds(c * CHUNK + g * LANES, LANES)] = v
            return 0

        lax.fori_loop(0, GROUPS, g_body, 0)

    def accum(l, pb):
        fr = frs[pb]
        fpb = fps[pb]

        def g_body(g, _):
            s = pl.ds(g * LANES, LANES)
            fx = fr[0, s]
            fy = fr[1, s]
            fz = fr[2, s]
            gx = 1.0 - fx
            gy = 1.0 - fy
            gz = 1.0 - fz
            a0 = jnp.zeros((LANES,), jnp.float32)
            a1 = jnp.zeros((LANES,), jnp.float32)
            for c in range(8):
                w = ((fx if c & 1 else gx) * (fy if c & 2 else gy)
                     * (fz if c & 4 else gz))
                pk = fpb[pl.ds(c * CHUNK + g * LANES, LANES)]
                f0v, f1v = plsc.unpack(plsc.bitcast(pk, jnp.bfloat16),
                                       format=plsc.PackFormat.INTERLEAVED)
                a0 = a0 + w * f0v
                a1 = a1 + w * f1v
            acc[2 * l, s] = a0
            acc[2 * l + 1, s] = a1
            return 0

        lax.fori_loop(0, GROUPS, g_body, 0)

    def dense_all():
        def g_body(g, _):
            s = pl.ds(g * LANES, LANES)
            x0 = xyz[0, s]
            y0 = xyz[1, s]
            z0 = xyz[2, s]
            for l in range(N_LOCAL):
                scale, res, _ = LEVELS[l]
                px = x0 * scale + 0.5
                py = y0 * scale + 0.5
                pz = z0 * scale + 0.5
                ix = px.astype(jnp.int32)
                iy = py.astype(jnp.int32)
                iz = pz.astype(jnp.int32)
                fx = px - ix.astype(jnp.float32)
                fy = py - iy.astype(jnp.float32)
                fz = pz - iz.astype(jnp.float32)
                gx = 1.0 - fx
                gy = 1.0 - fy
                gz = 1.0 - fz
                hx = (ix + DOFF[l], ix + (DOFF[l] + 1))
                hy = (iy * res, iy * res + res)
                hz = (iz * (res * res), iz * (res * res) + res * res)
                a0 = jnp.zeros((LANES,), jnp.float32)
                a1 = jnp.zeros((LANES,), jnp.float32)
                for c in range(8):
                    w = ((fx if c & 1 else gx) * (fy if c & 2 else gy)
                         * (fz if c & 4 else gz))
                    iv = hx[c & 1] + hy[(c >> 1) & 1] + hz[(c >> 2) & 1]
                    pk = plsc.load_gather(dloc, [iv])
                    f0v, f1v = plsc.unpack(plsc.bitcast(pk, jnp.bfloat16),
                                           format=plsc.PackFormat.INTERLEAVED)
                    a0 = a0 + w * f0v
                    a1 = a1 + w * f1v
                acc[2 * l, s] = a0
                acc[2 * l + 1, s] = a1
            return 0

        lax.fori_loop(0, GROUPS, g_body, 0)

    def issue(l, pb):
        src = shtab if l in SH_OFF else tbl
        return pltpu.async_copy(src.at[idxs[pb]], fps[pb], sems[pb])

    def chunk_body(ci, _):
        cbase = base + ci * CHUNK
        pltpu.sync_copy(xT.at[:, pl.ds(cbase, CHUNK)], xyz)
        desc = [None, None]
        compute_idx(DMA_LEVELS[0], 0)
        desc[0] = issue(DMA_LEVELS[0], 0)
        for i in range(1, len(DMA_LEVELS)):
            pb = i % 2
            compute_idx(DMA_LEVELS[i], pb)
            desc[pb] = issue(DMA_LEVELS[i], pb)
            if i == 1:
                dense_all()
            desc[1 - pb].wait()
            accum(DMA_LEVELS[i - 1], 1 - pb)
        last = (len(DMA_LEVELS) - 1) % 2
        desc[last].wait()
        accum(DMA_LEVELS[-1], last)
        pltpu.sync_copy(acc, out.at[:, pl.ds(cbase, CHUNK)])
        return 0

    lax.fori_loop(0, NCHUNK, chunk_body, 0)


@functools.partial(jax.jit, static_argnames=())
def _encode(xT, tbl, dense, sh):
    kern = pl.kernel(
        _encode_body,
        out_type=jax.ShapeDtypeStruct((2 * N_LEVELS, N_POINTS), jnp.float32),
        mesh=plsc.VectorSubcoreMesh(core_axis_name="c", subcore_axis_name="s"),
        scratch_types=[
            pltpu.VMEM((3, CHUNK), jnp.float32),
            pltpu.VMEM((DENSE_TOTAL,), jnp.float32),
            pltpu.VMEM_SHARED((SH_TOTAL,), jnp.float32),
            pltpu.VMEM((3, CHUNK), jnp.float32),
            pltpu.VMEM((3, CHUNK), jnp.float32),
            pltpu.VMEM((NIDX,), jnp.int32),
            pltpu.VMEM((NIDX,), jnp.int32),
            pltpu.VMEM((NIDX,), jnp.float32),
            pltpu.VMEM((NIDX,), jnp.float32),
            pltpu.VMEM((2 * N_LEVELS, CHUNK), jnp.float32),
            pltpu.SemaphoreType.DMA,
            pltpu.SemaphoreType.DMA,
        ],
        compiler_params=pltpu.CompilerParams(
            needs_layout_passes=False, use_tc_tiling_on_sc=False),
    )
    return kern(xT, tbl, dense, sh)


def _softplus10(v):
    t = 10.0 * v
    return (jnp.maximum(t, 0.0) + jnp.log1p(jnp.exp(-jnp.abs(t)))) * 0.1


BN = 4096


def _mlp_body(e_ref, w0_ref, w1_ref, w2_ref, w3_ref, o_ref):
    blk = e_ref[...]
    h = _softplus10(jnp.dot(w0_ref[...], blk, preferred_element_type=jnp.float32))
    h = _softplus10(jnp.dot(w1_ref[...], h, preferred_element_type=jnp.float32))
    h = _softplus10(jnp.dot(w2_ref[...], h, preferred_element_type=jnp.float32))
    o_ref[...] = jnp.dot(w3_ref[...], h, preferred_element_type=jnp.float32)


def _mlp(enc, W0, W1, W2, W3):
    grid = (N_POINTS // BN,)
    return pl.pallas_call(
        _mlp_body,
        grid=grid,
        in_specs=[
            pl.BlockSpec((2 * N_LEVELS, BN), lambda i: (0, i)),
            pl.BlockSpec(W0.shape, lambda i: (0, 0)),
            pl.BlockSpec(W1.shape, lambda i: (0, 0)),
            pl.BlockSpec(W2.shape, lambda i: (0, 0)),
            pl.BlockSpec(W3.shape, lambda i: (0, 0)),
        ],
        out_specs=pl.BlockSpec((1, BN), lambda i: (0, i)),
        out_shape=jax.ShapeDtypeStruct((1, N_POINTS), jnp.float32),
    )(enc, W0, W1, W2, W3)


def kernel(x, table, W0, W1, W2, W3):
    xT = x.T
    tbl = jax.lax.bitcast_convert_type(
        table.astype(jnp.bfloat16).reshape(N_LEVELS * T_SIZE, F_DIM),
        jnp.float32)
    dense = jnp.concatenate(
        [tbl[l * T_SIZE:l * T_SIZE + _DN[l]] for l in range(N_LOCAL)]
        + [jnp.zeros((DENSE_TOTAL - sum(_DN),), jnp.float32)])
    sh = jnp.concatenate(
        [tbl[l * T_SIZE:l * T_SIZE + _sh_size(l)] for l in SH_LEVELS]
        + [jnp.zeros((SH_TOTAL - _acc,), jnp.float32)])
    enc = _encode(xT, tbl, dense, sh)
    out = _mlp(enc, W0, W1, W2, W3)
    return out.reshape(N_POINTS, 1)
